# baseline (speedup 1.0000x reference)
import jax
import jax.numpy as jnp
from jax import lax
from jax.experimental import pallas as pl
from jax.experimental.pallas import tpu as pltpu
from jax.experimental.pallas import tpu_sc as plsc

NUM_CONT = 13
NUM_CAT = 26
VOCAB1 = 100001
D = 16
B = 16384
EMB_W = NUM_CAT * D
OUT_W = NUM_CONT + EMB_W

NC = 2
NS = 16
NW = NC * NS
BPW = B // NW
SB = 128
NSUB = BPW // SB
G = 128
IDX_PER_W = BPW * NUM_CAT
NGRP = IDX_PER_W // G
NG_SUB = SB * NUM_CAT // G


def _body(xcat_hbm, tab_hbm, out_hbm, idx_v, rows_v, sem):
  wid = lax.axis_index("s") * NC + lax.axis_index("c")

  pltpu.sync_copy(xcat_hbm.at[pl.ds(wid * NGRP, NGRP)], idx_v)

  lanes = lax.iota(jnp.int32, 16)

  @pl.loop(0, NGRP)
  def _idx_loop(g):
    for k in range(G // 16):
      j = g * G + k * 16 + lanes
      sl = idx_v.at[g, pl.ds(k * 16, 16)]
      sl[...] = sl[...] + (j % NUM_CAT) * VOCAB1

  for t in range(NSUB):

    @pl.loop(0, NG_SUB)
    def _fire(g):
      pltpu.make_async_copy(
          tab_hbm.at[idx_v.at[t * NG_SUB + g]],
          rows_v.at[pl.ds(g * G, G)],
          sem,
      ).start()

    @pl.loop(0, NG_SUB)
    def _drain(g):
      pltpu.make_async_copy(
          tab_hbm.at[idx_v.at[0]],
          rows_v.at[pl.ds(0, G)],
          sem,
      ).wait()

    pltpu.sync_copy(
        rows_v,
        out_hbm.at[pl.ds(wid * IDX_PER_W + t * SB * NUM_CAT, SB * NUM_CAT)])


@jax.jit
def _run(xcat, tab_flat):
  mesh = plsc.VectorSubcoreMesh(
      core_axis_name="c", subcore_axis_name="s", num_cores=NC)
  f = pl.kernel(
      _body,
      out_type=jax.ShapeDtypeStruct((B * NUM_CAT, D), jnp.float32),
      mesh=mesh,
      compiler_params=pltpu.CompilerParams(use_tc_tiling_on_sc=False),
      scratch_types=[
          pltpu.VMEM((NGRP, G), jnp.int32),
          pltpu.VMEM((SB * NUM_CAT, D), jnp.float32),
          pltpu.SemaphoreType.DMA,
      ],
  )
  return f(xcat, tab_flat)


def kernel(input_x, tables):
  x = input_x.astype(jnp.int32)
  xcat = x[:, NUM_CONT:].reshape(NW * NGRP, G)
  tab_flat = tables.reshape(NUM_CAT * VOCAB1, D)
  emb = _run(xcat, tab_flat)
  cont = x[:, :NUM_CONT].astype(jnp.float32)
  return jnp.concatenate([cont, emb.reshape(B, EMB_W)], axis=1)


# speedup vs baseline: 1.9568x; 1.9568x over previous
import jax
import jax.numpy as jnp
from jax import lax
from jax.experimental import pallas as pl
from jax.experimental.pallas import tpu as pltpu
from jax.experimental.pallas import tpu_sc as plsc

NUM_CONT = 13
NUM_CAT = 26
VOCAB1 = 100001
D = 16
B = 16384
OUT_W = NUM_CONT + NUM_CAT * D

NC = 2
NS = 16
NW = NC * NS
BPW = B // NW
SB = 128
NSUB = BPW // SB


def _body(xcatT_hbm, cont_hbm, tab_hbm, out_hbm, idx_v, rows_v, out_v, sem):
  wid = lax.axis_index("s") * NC + lax.axis_index("c")
  base = wid * BPW

  pltpu.sync_copy(xcatT_hbm.at[:, pl.ds(base, BPW)], idx_v)

  for t in range(NSUB):
    row0 = base + t * SB

    cont_cp = pltpu.make_async_copy(
        cont_hbm.at[pl.ds(row0, SB)], out_v.at[:, pl.ds(0, D)], sem)
    cont_cp.start()

    @pl.loop(0, NUM_CAT)
    def _fire(f):
      pltpu.make_async_copy(
          tab_hbm.at[f].at[idx_v.at[f, pl.ds(t * SB, SB)]],
          rows_v.at[pl.ds(f * SB, SB)],
          sem,
      ).start()

    cont_cp.wait()

    @pl.loop(0, NUM_CAT)
    def _drain(f):
      pltpu.make_async_copy(
          tab_hbm.at[0].at[idx_v.at[0, pl.ds(0, SB)]],
          rows_v.at[pl.ds(0, SB)],
          sem,
      ).wait()

    for f in range(NUM_CAT):

      @pl.loop(0, SB, unroll=4)
      def _repack(j):
        out_v[j, pl.ds(NUM_CONT + D * f, D)] = rows_v[f * SB + j]

    pltpu.sync_copy(out_v, out_hbm.at[pl.ds(row0, SB)])


@jax.jit
def _run(xcatT, cont, tab):
  mesh = plsc.VectorSubcoreMesh(
      core_axis_name="c", subcore_axis_name="s", num_cores=NC)
  f = pl.kernel(
      _body,
      out_type=jax.ShapeDtypeStruct((B, OUT_W), jnp.float32),
      mesh=mesh,
      compiler_params=pltpu.CompilerParams(use_tc_tiling_on_sc=False),
      scratch_types=[
          pltpu.VMEM((NUM_CAT, BPW), jnp.int32),
          pltpu.VMEM((NUM_CAT * SB, D), jnp.float32),
          pltpu.VMEM((SB, OUT_W), jnp.float32),
          pltpu.SemaphoreType.DMA,
      ],
  )
  return f(xcatT, cont, tab)


def kernel(input_x, tables):
  x = input_x.astype(jnp.int32)
  xcatT = x[:, NUM_CONT:].T
  cont = jnp.pad(
      x[:, :NUM_CONT].astype(jnp.float32), ((0, 0), (0, D - NUM_CONT)))
  return _run(xcatT, cont, tables)
